# baseline (speedup 1.0000x reference)
import functools

import jax
import jax.numpy as jnp
from jax.experimental import pallas as pl
from jax.experimental.pallas import tpu as pltpu

GC_LAMBDA = 0.5
N_ITERS = 5
H = 512
W = 512
NB = 8
BR = H // NB


def _nsum(x, zero_row, zero_col):
    up = jnp.concatenate([x[1:, :], zero_row], axis=0)
    down = jnp.concatenate([zero_row, x[:-1, :]], axis=0)
    left = jnp.concatenate([x[:, 1:], zero_col], axis=1)
    right = jnp.concatenate([zero_col, x[:, :-1]], axis=1)
    return (up + down) + (left + right)


def _sweep_band(xext, c_band, half_row_b, half_col_b, one_b, zero_b):
    x = xext[1:BR + 1, :]
    up = xext[2:BR + 2, :]
    down = xext[0:BR, :]
    left = jnp.concatenate([x[:, 1:], half_col_b], axis=1)
    right = jnp.concatenate([half_col_b, x[:, :-1]], axis=1)
    s = (up + down) + (left + right)
    return jnp.minimum(jnp.maximum((s + s) - c_band, zero_b), one_b)


def _icm_kernel(p_hbm, out_hbm, p_vmem, out_vmem, c_ref, lab_ref, lab2_ref,
                sem_in, sem_out):
    half_row = jnp.full((1, W), 0.5, jnp.bfloat16)
    half_col_b = jnp.full((BR, 1), 0.5, jnp.bfloat16)
    one_b = jnp.full((BR, W), 1.0, jnp.bfloat16)
    zero_b = jnp.zeros((BR, W), jnp.bfloat16)

    for b in range(NB):
        pltpu.make_async_copy(
            p_hbm.at[pl.ds(BR * b, BR)], p_vmem.at[pl.ds(BR * b, BR)],
            sem_in.at[b]).start()

    def sweep1_band(bs):
        r0 = BR * bs
        if bs == 0:
            xext = jnp.concatenate(
                [half_row, lab_ref[pl.ds(0, BR + 1), :]], axis=0)
        elif bs == NB - 1:
            xext = jnp.concatenate(
                [lab_ref[pl.ds(r0 - 1, BR + 1), :], half_row], axis=0)
        else:
            xext = lab_ref[pl.ds(r0 - 1, BR + 2), :]
        c_band = c_ref[pl.ds(r0, BR), :]
        lab2_ref[pl.ds(r0, BR), :] = _sweep_band(
            xext, c_band, half_row, half_col_b, one_b, zero_b)

    for b in range(NB):
        pltpu.make_async_copy(
            p_hbm.at[pl.ds(BR * b, BR)], p_vmem.at[pl.ds(BR * b, BR)],
            sem_in.at[b]).wait()
        p = p_vmem[pl.ds(BR * b, BR), :]
        d2 = 2.0 * jnp.log((1.0 - p) / p)
        c_ref[pl.ds(BR * b, BR), :] = (
            jnp.clip(jnp.floor(d2), -5.0, 4.0) + 4.0).astype(jnp.bfloat16)
        lab_ref[pl.ds(BR * b, BR), :] = (p > 0.5).astype(jnp.bfloat16)
        if b >= 1:
            sweep1_band(b - 1)
    sweep1_band(NB - 1)

    c = c_ref[...]
    half_col = jnp.full((H, 1), 0.5, jnp.bfloat16)
    one = jnp.full((H, W), 1.0, jnp.bfloat16)
    zero = jnp.zeros((H, W), jnp.bfloat16)
    labels = lab2_ref[...]
    for _ in range(N_ITERS - 2):
        s = _nsum(labels, half_row, half_col)
        labels = jnp.minimum(jnp.maximum((s + s) - c, zero), one)

    for b in range(NB):
        r0 = BR * b
        if b == 0:
            xext = jnp.concatenate([half_row, labels[:BR + 1, :]], axis=0)
        elif b == NB - 1:
            xext = jnp.concatenate([labels[r0 - 1:, :], half_row], axis=0)
        else:
            xext = labels[r0 - 1:r0 + BR + 1, :]
        lab_b = _sweep_band(xext, c[r0:r0 + BR, :], half_row, half_col_b,
                            one_b, zero_b)
        out_vmem[pl.ds(r0, BR), :] = lab_b.astype(jnp.float32)
        pltpu.make_async_copy(
            out_vmem.at[pl.ds(r0, BR)], out_hbm.at[pl.ds(r0, BR)],
            sem_out.at[b]).start()

    for b in range(NB):
        pltpu.make_async_copy(
            out_vmem.at[pl.ds(BR * b, BR)], out_hbm.at[pl.ds(BR * b, BR)],
            sem_out.at[b]).wait()


@jax.jit
def kernel(probs):
    out = pl.pallas_call(
        _icm_kernel,
        out_shape=jax.ShapeDtypeStruct((H, W), jnp.float32),
        in_specs=[pl.BlockSpec(memory_space=pl.ANY)],
        out_specs=pl.BlockSpec(memory_space=pl.ANY),
        scratch_shapes=[
            pltpu.VMEM((H, W), jnp.float32),
            pltpu.VMEM((H, W), jnp.float32),
            pltpu.VMEM((H, W), jnp.bfloat16),
            pltpu.VMEM((H, W), jnp.bfloat16),
            pltpu.VMEM((H, W), jnp.bfloat16),
            pltpu.SemaphoreType.DMA((NB,)),
            pltpu.SemaphoreType.DMA((NB,)),
        ],
    )(probs.reshape(H, W))
    return out.reshape(1, H, W)


# speedup vs baseline: 1.0984x; 1.0984x over previous
import functools

import jax
import jax.numpy as jnp
from jax.experimental import pallas as pl
from jax.experimental.pallas import tpu as pltpu

GC_LAMBDA = 0.5
N_ITERS = 5
H = 512
W = 512
NB = 8
BR = H // NB


def _nsum(x, zero_row, zero_col):
    up = jnp.concatenate([x[1:, :], zero_row], axis=0)
    down = jnp.concatenate([zero_row, x[:-1, :]], axis=0)
    left = jnp.concatenate([x[:, 1:], zero_col], axis=1)
    right = jnp.concatenate([zero_col, x[:, :-1]], axis=1)
    return (up + down) + (left + right)


def _icm_kernel(p_hbm, out_hbm, p_vmem, out_vmem, sem_in, sem_out):
    pltpu.make_async_copy(p_hbm, p_vmem, sem_in).start()
    pltpu.make_async_copy(p_hbm, p_vmem, sem_in).wait()
    p = p_vmem[...]
    d2 = 2.0 * jnp.log((1.0 - p) / p)
    c = (jnp.clip(jnp.floor(d2), -5.0, 4.0) + 4.0).astype(jnp.bfloat16)
    half_row = jnp.full((1, W), 0.5, jnp.bfloat16)
    half_col = jnp.full((H, 1), 0.5, jnp.bfloat16)
    one = jnp.ones((H, W), jnp.bfloat16)
    zero = jnp.zeros((H, W), jnp.bfloat16)
    labels = (p > 0.5).astype(jnp.bfloat16)
    for _ in range(N_ITERS - 1):
        s = _nsum(labels, half_row, half_col)
        labels = jnp.minimum(jnp.maximum((s + s) - c, zero), one)

    half_col_b = jnp.full((BR, 1), 0.5, jnp.bfloat16)
    for b in range(NB):
        r0 = BR * b
        xb = labels[r0:r0 + BR, :]
        if b == NB - 1:
            up = jnp.concatenate([labels[r0 + 1:, :], half_row], axis=0)
        else:
            up = labels[r0 + 1:r0 + BR + 1, :]
        if b == 0:
            down = jnp.concatenate([half_row, labels[:BR - 1, :]], axis=0)
        else:
            down = labels[r0 - 1:r0 + BR - 1, :]
        left = jnp.concatenate([xb[:, 1:], half_col_b], axis=1)
        right = jnp.concatenate([half_col_b, xb[:, :-1]], axis=1)
        s = (up + down) + (left + right)
        lab_b = jnp.minimum(jnp.maximum((s + s) - c[r0:r0 + BR, :],
                                        zero[:BR, :]), one[:BR, :])
        out_vmem[pl.ds(r0, BR), :] = lab_b.astype(jnp.float32)
        pltpu.make_async_copy(
            out_vmem.at[pl.ds(r0, BR)], out_hbm.at[pl.ds(r0, BR)],
            sem_out.at[b]).start()

    for b in range(NB):
        pltpu.make_async_copy(
            out_vmem.at[pl.ds(BR * b, BR)], out_hbm.at[pl.ds(BR * b, BR)],
            sem_out.at[b]).wait()


@jax.jit
def kernel(probs):
    out = pl.pallas_call(
        _icm_kernel,
        out_shape=jax.ShapeDtypeStruct((H, W), jnp.float32),
        in_specs=[pl.BlockSpec(memory_space=pl.ANY)],
        out_specs=pl.BlockSpec(memory_space=pl.ANY),
        scratch_shapes=[
            pltpu.VMEM((H, W), jnp.float32),
            pltpu.VMEM((H, W), jnp.float32),
            pltpu.SemaphoreType.DMA,
            pltpu.SemaphoreType.DMA((NB,)),
        ],
    )(probs.reshape(H, W))
    return out.reshape(1, H, W)


# speedup vs baseline: 1.1094x; 1.0100x over previous
import functools

import jax
import jax.numpy as jnp
from jax.experimental import pallas as pl
from jax.experimental.pallas import tpu as pltpu

GC_LAMBDA = 0.5
N_ITERS = 5
H = 512
W = 512
NB = 8
BR = H // NB


def _nsum(x, zero_row, zero_col):
    up = jnp.concatenate([x[1:, :], zero_row], axis=0)
    down = jnp.concatenate([zero_row, x[:-1, :]], axis=0)
    left = jnp.concatenate([x[:, 1:], zero_col], axis=1)
    right = jnp.concatenate([zero_col, x[:, :-1]], axis=1)
    return (up + down) + (left + right)


def _icm_kernel(p_hbm, out_hbm, p_vmem, out_vmem, c_ref, lab_ref, sem_in, sem_out):
    for b in range(NB):
        pltpu.make_async_copy(
            p_hbm.at[pl.ds(BR * b, BR)], p_vmem.at[pl.ds(BR * b, BR)],
            sem_in.at[b]).start()

    for b in range(NB):
        pltpu.make_async_copy(
            p_hbm.at[pl.ds(BR * b, BR)], p_vmem.at[pl.ds(BR * b, BR)],
            sem_in.at[b]).wait()
        p = p_vmem[pl.ds(BR * b, BR), :]
        d2 = 2.0 * jnp.log((1.0 - p) / p)
        cb = (jnp.clip(jnp.floor(d2), -5.0, 4.0) + 4.0).astype(jnp.bfloat16)
        c_ref[pl.ds(BR * b, BR), :] = cb
        lab_ref[pl.ds(BR * b, BR), :] = jnp.minimum(
            jnp.maximum(jnp.bfloat16(4.0) - cb, jnp.bfloat16(0.0)),
            jnp.bfloat16(1.0))

    c = c_ref[...]
    half_row = jnp.full((1, W), 0.5, jnp.bfloat16)
    half_col = jnp.full((H, 1), 0.5, jnp.bfloat16)
    one = jnp.ones((H, W), jnp.bfloat16)
    zero = jnp.zeros((H, W), jnp.bfloat16)
    labels = lab_ref[...]
    for _ in range(N_ITERS - 1):
        s = _nsum(labels, half_row, half_col)
        labels = jnp.minimum(jnp.maximum((s + s) - c, zero), one)

    half_col_b = jnp.full((BR, 1), 0.5, jnp.bfloat16)
    for b in range(NB):
        r0 = BR * b
        xb = labels[r0:r0 + BR, :]
        if b == NB - 1:
            up = jnp.concatenate([labels[r0 + 1:, :], half_row], axis=0)
        else:
            up = labels[r0 + 1:r0 + BR + 1, :]
        if b == 0:
            down = jnp.concatenate([half_row, labels[:BR - 1, :]], axis=0)
        else:
            down = labels[r0 - 1:r0 + BR - 1, :]
        left = jnp.concatenate([xb[:, 1:], half_col_b], axis=1)
        right = jnp.concatenate([half_col_b, xb[:, :-1]], axis=1)
        s = (up + down) + (left + right)
        lab_b = jnp.minimum(jnp.maximum((s + s) - c[r0:r0 + BR, :],
                                        zero[:BR, :]), one[:BR, :])
        out_vmem[pl.ds(r0, BR), :] = lab_b.astype(jnp.float32)
        pltpu.make_async_copy(
            out_vmem.at[pl.ds(r0, BR)], out_hbm.at[pl.ds(r0, BR)],
            sem_out.at[b]).start()

    for b in range(NB):
        pltpu.make_async_copy(
            out_vmem.at[pl.ds(BR * b, BR)], out_hbm.at[pl.ds(BR * b, BR)],
            sem_out.at[b]).wait()


@jax.jit
def kernel(probs):
    out = pl.pallas_call(
        _icm_kernel,
        out_shape=jax.ShapeDtypeStruct((H, W), jnp.float32),
        in_specs=[pl.BlockSpec(memory_space=pl.ANY)],
        out_specs=pl.BlockSpec(memory_space=pl.ANY),
        scratch_shapes=[
            pltpu.VMEM((H, W), jnp.float32),
            pltpu.VMEM((H, W), jnp.float32),
            pltpu.VMEM((H, W), jnp.bfloat16),
            pltpu.VMEM((H, W), jnp.bfloat16),
            pltpu.SemaphoreType.DMA((NB,)),
            pltpu.SemaphoreType.DMA((NB,)),
        ],
    )(probs.reshape(H, W))
    return out.reshape(1, H, W)


# speedup vs baseline: 1.1174x; 1.0073x over previous
import functools

import jax
import jax.numpy as jnp
from jax.experimental import pallas as pl
from jax.experimental.pallas import tpu as pltpu

GC_LAMBDA = 0.5
N_ITERS = 5
H = 512
W = 512
NB = 8
BR = H // NB


def _nsum(x, zero_row, zero_col):
    up = jnp.concatenate([x[1:, :], zero_row], axis=0)
    down = jnp.concatenate([zero_row, x[:-1, :]], axis=0)
    left = jnp.concatenate([x[:, 1:], zero_col], axis=1)
    right = jnp.concatenate([zero_col, x[:, :-1]], axis=1)
    return (up + down) + (left + right)


def _icm_kernel(p_hbm, out_hbm, p_vmem, out_vmem, c_ref, lab_ref, sem_in, sem_out):
    for b in range(NB):
        pltpu.make_async_copy(
            p_hbm.at[pl.ds(BR * b, BR)], p_vmem.at[pl.ds(BR * b, BR)],
            sem_in.at[b]).start()

    for b in range(NB):
        pltpu.make_async_copy(
            p_hbm.at[pl.ds(BR * b, BR)], p_vmem.at[pl.ds(BR * b, BR)],
            sem_in.at[b]).wait()
        p = p_vmem[pl.ds(BR * b, BR), :]
        d2 = 2.0 * jnp.log((1.0 - p) / p)
        c_ref[pl.ds(BR * b, BR), :] = (
            jnp.clip(jnp.floor(d2), -5.0, 4.0) + 4.0).astype(jnp.bfloat16)
        lab_ref[pl.ds(BR * b, BR), :] = (p > 0.5).astype(jnp.bfloat16)

    c = c_ref[...]
    half_row = jnp.full((1, W), 0.5, jnp.bfloat16)
    half_col = jnp.full((H, 1), 0.5, jnp.bfloat16)
    one = jnp.ones((H, W), jnp.bfloat16)
    zero = jnp.zeros((H, W), jnp.bfloat16)
    labels = lab_ref[...]
    for _ in range(N_ITERS - 1):
        s = _nsum(labels, half_row, half_col)
        labels = jnp.minimum(jnp.maximum((s + s) - c, zero), one)

    half_col_b = jnp.full((BR, 1), 0.5, jnp.bfloat16)
    for b in range(NB):
        r0 = BR * b
        xb = labels[r0:r0 + BR, :]
        if b == NB - 1:
            up = jnp.concatenate([labels[r0 + 1:, :], half_row], axis=0)
        else:
            up = labels[r0 + 1:r0 + BR + 1, :]
        if b == 0:
            down = jnp.concatenate([half_row, labels[:BR - 1, :]], axis=0)
        else:
            down = labels[r0 - 1:r0 + BR - 1, :]
        left = jnp.concatenate([xb[:, 1:], half_col_b], axis=1)
        right = jnp.concatenate([half_col_b, xb[:, :-1]], axis=1)
        s = (up + down) + (left + right)
        lab_b = jnp.minimum(jnp.maximum((s + s) - c[r0:r0 + BR, :],
                                        zero[:BR, :]), one[:BR, :])
        out_vmem[pl.ds(r0, BR), :] = lab_b.astype(jnp.float32)
        pltpu.make_async_copy(
            out_vmem.at[pl.ds(r0, BR)], out_hbm.at[pl.ds(r0, BR)],
            sem_out.at[b]).start()

    for b in range(NB):
        pltpu.make_async_copy(
            out_vmem.at[pl.ds(BR * b, BR)], out_hbm.at[pl.ds(BR * b, BR)],
            sem_out.at[b]).wait()


@jax.jit
def kernel(probs):
    out = pl.pallas_call(
        _icm_kernel,
        out_shape=jax.ShapeDtypeStruct((H, W), jnp.float32),
        in_specs=[pl.BlockSpec(memory_space=pl.ANY)],
        out_specs=pl.BlockSpec(memory_space=pl.ANY),
        scratch_shapes=[
            pltpu.VMEM((H, W), jnp.float32),
            pltpu.VMEM((H, W), jnp.float32),
            pltpu.VMEM((H, W), jnp.bfloat16),
            pltpu.VMEM((H, W), jnp.bfloat16),
            pltpu.SemaphoreType.DMA((NB,)),
            pltpu.SemaphoreType.DMA((NB,)),
        ],
    )(probs.reshape(H, W))
    return out.reshape(1, H, W)
